# baseline (speedup 1.0000x reference)
import functools

import jax
import jax.numpy as jnp
from jax import lax
from jax.experimental import pallas as pl
from jax.experimental.pallas import tpu as pltpu
from jax.experimental.pallas import tpu_sc as plsc

B = 1024
N = 200
K = 50
D = 128

NC = 2
NS = 16
NW = NC * NS
ROWS = B * K
RPW = ROWS // NW
CH = 80
NCH = RPW // CH
L = 16


def _build_sc_gather():
    mesh = plsc.VectorSubcoreMesh(core_axis_name="c", subcore_axis_name="s")

    @functools.partial(
        pl.kernel,
        mesh=mesh,
        out_type=jax.ShapeDtypeStruct((ROWS, D), jnp.float32),
        scratch_types=[
            pltpu.VMEM((RPW,), jnp.int32),
            pltpu.VMEM((RPW,), jnp.int32),
            pltpu.VMEM((NCH, CH), jnp.int32),
            pltpu.VMEM((CH, D), jnp.float32),
            pltpu.VMEM((CH, D), jnp.float32),
            pltpu.SemaphoreType.DMA,
            pltpu.SemaphoreType.DMA,
        ],
    )
    def sc_gather(table_hbm, idx_hbm, off_hbm, out_hbm,
                  raw_v, off_v, flat_v, rows0, rows1, sem0, sem1):
        wid = lax.axis_index("s") * NC + lax.axis_index("c")
        base = wid * RPW

        pltpu.sync_copy(idx_hbm.at[pl.ds(base, RPW)], raw_v)
        pltpu.sync_copy(off_hbm.at[pl.ds(base, RPW)], off_v)

        for j in range(NCH):
            for c in range(CH // L):
                r0 = j * CH + c * L
                flat_v[j, pl.ds(c * L, L)] = (
                    raw_v[pl.ds(r0, L)] + off_v[pl.ds(r0, L)])

        bufs = (rows0, rows1)
        sems = (sem0, sem1)
        pending = pltpu.async_copy(table_hbm.at[flat_v.at[0]], bufs[0], sems[0])
        for j in range(NCH):
            nxt = None
            if j + 1 < NCH:
                nxt = pltpu.async_copy(
                    table_hbm.at[flat_v.at[j + 1]],
                    bufs[(j + 1) % 2], sems[(j + 1) % 2])
            pending.wait()
            pltpu.sync_copy(bufs[j % 2], out_hbm.at[pl.ds(base + j * CH, CH)])
            pending = nxt

    return sc_gather


_sc_gather = _build_sc_gather()


def kernel(input_tensor, indices):
    table = input_tensor.reshape(B * N, D)
    idx = indices.astype(jnp.int32).reshape(ROWS)
    off = jnp.repeat(jnp.arange(B, dtype=jnp.int32) * N, K, total_repeat_length=ROWS)
    out = _sc_gather(table, idx, off)
    return out.reshape(B, K, D)


# speedup vs baseline: 1.0049x; 1.0049x over previous
import functools

import jax
import jax.numpy as jnp
from jax import lax
from jax.experimental import pallas as pl
from jax.experimental.pallas import tpu as pltpu
from jax.experimental.pallas import tpu_sc as plsc

B = 1024
N = 200
K = 50
D = 128

NC = 2
NS = 16
NW = NC * NS
ROWS = B * K
RPW = ROWS // NW
CH = 80
NCH = RPW // CH
NBUF = 8
L = 16


def _build_sc_gather():
    mesh = plsc.VectorSubcoreMesh(core_axis_name="c", subcore_axis_name="s")

    @functools.partial(
        pl.kernel,
        mesh=mesh,
        out_type=jax.ShapeDtypeStruct((ROWS, D), jnp.float32),
        scratch_types=[
            pltpu.VMEM((RPW,), jnp.int32),
            pltpu.VMEM((RPW,), jnp.int32),
            pltpu.VMEM((NCH, CH), jnp.int32),
        ] + [pltpu.VMEM((CH, D), jnp.float32) for _ in range(NBUF)]
          + [pltpu.SemaphoreType.DMA for _ in range(2 * NBUF)],
    )
    def sc_gather(table_hbm, idx_hbm, off_hbm, out_hbm,
                  raw_v, off_v, flat_v, *bufs_and_sems):
        bufs = bufs_and_sems[:NBUF]
        gsems = bufs_and_sems[NBUF:2 * NBUF]
        wsems = bufs_and_sems[2 * NBUF:]
        wid = lax.axis_index("s") * NC + lax.axis_index("c")
        base = wid * RPW

        pltpu.sync_copy(idx_hbm.at[pl.ds(base, RPW)], raw_v)
        pltpu.sync_copy(off_hbm.at[pl.ds(base, RPW)], off_v)

        for j in range(NCH):
            for c in range(CH // L):
                r0 = j * CH + c * L
                flat_v[j, pl.ds(c * L, L)] = (
                    raw_v[pl.ds(r0, L)] + off_v[pl.ds(r0, L)])

        gh = [None] * NBUF
        wh = [None] * NBUF
        for j in range(min(NBUF, NCH)):
            gh[j] = pltpu.async_copy(
                table_hbm.at[flat_v.at[j]], bufs[j], gsems[j])
        for j in range(NCH):
            b = j % NBUF
            gh[b].wait()
            wh[b] = pltpu.async_copy(
                bufs[b], out_hbm.at[pl.ds(base + j * CH, CH)], wsems[b])
            nj = j + NBUF
            if nj < NCH:
                wh[b].wait()
                gh[b] = pltpu.async_copy(
                    table_hbm.at[flat_v.at[nj]], bufs[b], gsems[b])
        for j in range(max(0, NCH - NBUF), NCH):
            wh[j % NBUF].wait()

    return sc_gather


_sc_gather = _build_sc_gather()


def kernel(input_tensor, indices):
    table = input_tensor.reshape(B * N, D)
    idx = indices.astype(jnp.int32).reshape(ROWS)
    off = jnp.repeat(jnp.arange(B, dtype=jnp.int32) * N, K, total_repeat_length=ROWS)
    out = _sc_gather(table, idx, off)
    return out.reshape(B, K, D)


# speedup vs baseline: 6.4972x; 6.4658x over previous
import functools

import jax
import jax.numpy as jnp
from jax import lax
from jax.experimental import pallas as pl
from jax.experimental.pallas import tpu as pltpu
from jax.experimental.pallas import tpu_sc as plsc

B = 1024
N = 200
K = 50
D = 128

NC = 2
NS = 16
NW = NC * NS
ROWS = B * K
RPW = ROWS // NW
CH = 80
NCH = RPW // CH
NBUF = 8
L = 16


def _build_sc_gather():
    mesh = plsc.VectorSubcoreMesh(core_axis_name="c", subcore_axis_name="s")

    @functools.partial(
        pl.kernel,
        mesh=mesh,
        out_type=jax.ShapeDtypeStruct((ROWS, D), jnp.float32),
        scratch_types=[
            pltpu.VMEM((RPW,), jnp.int32),
            pltpu.VMEM((RPW,), jnp.int32),
            pltpu.VMEM((NCH, CH), jnp.int32),
        ] + [pltpu.VMEM((CH, D), jnp.float32) for _ in range(NBUF)]
          + [pltpu.SemaphoreType.DMA for _ in range(2 * NBUF)],
    )
    def sc_gather(table_hbm, idx_hbm, off_hbm, out_hbm,
                  raw_v, off_v, flat_v, *bufs_and_sems):
        bufs = bufs_and_sems[:NBUF]
        gsems = bufs_and_sems[NBUF:2 * NBUF]
        wsems = bufs_and_sems[2 * NBUF:]
        wid = lax.axis_index("s") * NC + lax.axis_index("c")
        base = wid * RPW

        pltpu.sync_copy(idx_hbm.at[pl.ds(base, RPW)], raw_v)
        pltpu.sync_copy(off_hbm.at[pl.ds(base, RPW)], off_v)

        for j in range(NCH):
            for c in range(CH // L):
                r0 = j * CH + c * L
                flat_v[j, pl.ds(c * L, L)] = (
                    raw_v[pl.ds(r0, L)] + off_v[pl.ds(r0, L)])

        gh = [None] * NBUF
        wh = [None] * NBUF
        for j in range(min(NBUF, NCH)):
            gh[j] = pltpu.async_copy(
                table_hbm.at[flat_v.at[j]], bufs[j], gsems[j])
        for j in range(NCH):
            b = j % NBUF
            gh[b].wait()
            wh[b] = pltpu.async_copy(
                bufs[b], out_hbm.at[pl.ds(base + j * CH, CH)], wsems[b])
            nj = j + NBUF
            if nj < NCH:
                wh[b].wait()
                gh[b] = pltpu.async_copy(
                    table_hbm.at[flat_v.at[nj]], bufs[b], gsems[b])
        for j in range(max(0, NCH - NBUF), NCH):
            wh[j % NBUF].wait()

    return sc_gather


_sc_gather = _build_sc_gather()


def kernel(input_tensor, indices):
    table = input_tensor.reshape(B * N, D)
    idx = indices.astype(jnp.int32).reshape(ROWS)
    off = jnp.broadcast_to(
        (jnp.arange(B, dtype=jnp.int32) * N)[:, None], (B, K)).reshape(ROWS)
    out = _sc_gather(table, idx, off)
    return out.reshape(B, K, D)
